# baseline (speedup 1.0000x reference)
import functools

import jax
import jax.numpy as jnp
from jax import lax
from jax.experimental import pallas as pl
from jax.experimental.pallas import tpu as pltpu
from jax.experimental.pallas import tpu_sc as plsc

_NC = 2
_NS = 16
_NW = _NC * _NS
_CHUNK = 128
_BPW = 512
_NCH = _BPW // _CHUNK


def _repack_body(ina_ref, inb_ref, inc_ref, ind_ref, eye_ref, out_ref):
    e = eye_ref[...]
    dn = (((0,), (0,)), ((), ()))

    def t(ref):
        return lax.dot_general(ref[...], e, dn,
                               preferred_element_type=jnp.float32)

    def u32(x):
        b16 = lax.bitcast_convert_type(x.astype(jnp.bfloat16), jnp.uint16)
        return b16.astype(jnp.uint32)

    def pack(lo, hi):
        w = lax.bitwise_or(u32(lo), lax.shift_left(u32(hi), jnp.uint32(16)))
        return lax.bitcast_convert_type(w, jnp.float32)

    out_ref[:, 0:64] = pack(t(ina_ref), t(inc_ref))
    out_ref[:, 64:128] = pack(t(inb_ref), t(ind_ref))


def _repack(tab_t, eye, h, bk):
    n = tab_t.shape[1]
    grid = h // bk
    last_blk = (n - 1) // bk

    def inspec(k):
        return pl.BlockSpec(
            (64, bk),
            lambda ib, g=grid, lb=last_blk, k=k:
            (0, jnp.minimum(ib + k * g, lb)))

    return pl.pallas_call(
        _repack_body,
        grid=(grid,),
        in_specs=[inspec(0), inspec(1), inspec(2), inspec(3),
                  pl.BlockSpec((64, 64), lambda ib: (0, 0))],
        out_specs=pl.BlockSpec((bk, 128), lambda ib: (ib, 0)),
        out_shape=jax.ShapeDtypeStruct((h, 128), jnp.float32),
    )(tab_t, tab_t, tab_t, tab_t, eye)


def _sc_gather_body(n, *args):
    idxs = args[:n]
    pks = args[n:2 * n]
    outs_refs = args[2 * n:3 * n]
    idx_v, q_v, blocks_v, gsem, osem = args[3 * n:]
    wid = lax.axis_index("s") * _NC + lax.axis_index("c")
    row_base = wid * _BPW

    outs = []
    for t, (idx3, pk, out) in enumerate(zip(idxs, pks, outs_refs)):
        h = pk.shape[0]
        copies = []
        for c in range(_NCH):
            pltpu.sync_copy(idx3.at[wid * _NCH + c], idx_v)
            for j in range(_CHUNK // 16):
                v = idx_v[pl.ds(j * 16, 16)]
                q_v[c, pl.ds(j * 16, 16)] = lax.bitwise_and(v, h - 1)
            copies.append(pltpu.async_copy(
                pk.at[q_v.at[c]], blocks_v.at[pl.ds(c * _CHUNK, _CHUNK)],
                gsem))
        for cp in copies:
            cp.wait()
        outs.append(pltpu.async_copy(
            blocks_v, out.at[pl.ds(row_base, _BPW)], osem))
        outs[-1].wait()


def _sc_gather(idx_list, pk_list):
    n = len(idx_list)
    b = idx_list[0].shape[0]
    mesh = plsc.VectorSubcoreMesh(core_axis_name="c", subcore_axis_name="s")
    out_sds = jax.ShapeDtypeStruct((b, 128), jnp.float32)
    fn = pl.kernel(
        functools.partial(_sc_gather_body, n),
        out_type=(out_sds,) * n,
        mesh=mesh,
        scratch_types=[
            pltpu.VMEM((_CHUNK,), jnp.int32),
            pltpu.VMEM((_NCH, _CHUNK), jnp.int32),
            pltpu.VMEM((_BPW, 128), jnp.float32),
            pltpu.SemaphoreType.DMA,
            pltpu.SemaphoreType.DMA,
        ],
    )
    idx2 = [ix.reshape(b // _CHUNK, _CHUNK) for ix in idx_list]
    return fn(*idx2, *pk_list)


def _mlp_body(u_ref, g_ref, i_ref, su_ref, sg_ref, si_ref,
              w1u_ref, w1g_ref, w1i_ref, b1_ref,
              w2_ref, b2_ref, w3_ref, b3_ref, out_ref):
    col = lax.broadcasted_iota(jnp.int32, (1, 128), 1)
    low = col < 64

    def sel(pk_ref, s_ref):
        s = s_ref[...]
        w = lax.bitcast_convert_type(pk_ref[...], jnp.uint32)
        lo = lax.bitcast_convert_type(lax.shift_left(w, jnp.uint32(16)),
                                      jnp.float32)
        hi = lax.bitcast_convert_type(
            lax.bitwise_and(w, jnp.uint32(0xFFFF0000)), jnp.float32)
        v = jnp.where(s >= 1.5, hi, lo)
        odd = jnp.where(s >= 1.5, s - 2.0, s) >= 0.5
        keep = jnp.logical_xor(low, odd)
        return jnp.where(keep, v, 0.0)

    h = (sel(u_ref, su_ref) @ w1u_ref[...]
         + sel(g_ref, sg_ref) @ w1g_ref[...]
         + sel(i_ref, si_ref) @ w1i_ref[...]
         + b1_ref[...])
    h = jnp.maximum(h, 0.0)
    h2 = jnp.maximum(h @ w2_ref[...] + b2_ref[...], 0.0)
    o = h2 @ w3_ref[...] + b3_ref[...]
    out_ref[...] = 1.0 / (1.0 + jnp.exp(-o))


def _mlp(u, g, i, su, sg, si, W1, b1, W2, b2, W3, b3, blk=2048):
    b = u.shape[0]
    grid = b // blk
    w1u = jnp.concatenate([W1[0:64], W1[0:64]], axis=0)
    w1g = jnp.concatenate([W1[64:128], W1[64:128]], axis=0)
    w1i = jnp.concatenate([W1[128:192], W1[128:192]], axis=0)
    row_spec = pl.BlockSpec((blk, 128), lambda ib: (ib, 0))
    s_spec = pl.BlockSpec((blk, 1), lambda ib: (ib, 0))

    def full(a):
        return pl.BlockSpec(a.shape, lambda ib: (0,) * a.ndim)

    return pl.pallas_call(
        _mlp_body,
        grid=(grid,),
        in_specs=[row_spec, row_spec, row_spec,
                  s_spec, s_spec, s_spec,
                  full(w1u), full(w1g), full(w1i), full(b1),
                  full(W2), full(b2), full(W3), full(b3)],
        out_specs=pl.BlockSpec((blk, 1), lambda ib: (ib, 0)),
        out_shape=jax.ShapeDtypeStruct((b, 1), jnp.float32),
    )(u, g, i, su, sg, si, w1u, w1g, w1i, b1, W2, b2, W3, b3)


def kernel(user_input, genre_input, item_input, user_table, genre_table,
           item_table, W1, b1, W2, b2, W3, b3):
    b = user_input.shape[0]
    eye = jnp.eye(64, dtype=jnp.float32)
    upk = _repack(user_table.T, eye, 262144, 4096)
    (u,) = _sc_gather([user_input], [upk])
    ipk = _repack(item_table.T, eye, 262144, 4096)
    gpk = _repack(genre_table.T, eye, 256, 256)
    su = (user_input >> 18).astype(jnp.float32).reshape(b, 1)
    si = (item_input >> 18).astype(jnp.float32).reshape(b, 1)
    sg = (genre_input >> 8).astype(jnp.float32).reshape(b, 1)
    g, i = _sc_gather([genre_input, item_input], [gpk, ipk])
    return _mlp(u, g, i, su, sg, si, W1, b1, W2, b2, W3, b3)


# speedup vs baseline: 1.0341x; 1.0341x over previous
import functools

import jax
import jax.numpy as jnp
from jax import lax
from jax.experimental import pallas as pl
from jax.experimental.pallas import tpu as pltpu
from jax.experimental.pallas import tpu_sc as plsc

_NC = 2
_NS = 16
_NW = _NC * _NS
_CHUNK = 128
_BPW = 512
_NCH = _BPW // _CHUNK


def _repack_body(ina_ref, inb_ref, inc_ref, ind_ref, eye_ref, out_ref):
    e = eye_ref[...]
    dn = (((0,), (0,)), ((), ()))

    def t(ref):
        return lax.dot_general(ref[...], e, dn,
                               preferred_element_type=jnp.float32)

    def u32(x):
        b16 = lax.bitcast_convert_type(x.astype(jnp.bfloat16), jnp.uint16)
        return b16.astype(jnp.uint32)

    def pack(lo, hi):
        w = lax.bitwise_or(u32(lo), lax.shift_left(u32(hi), jnp.uint32(16)))
        return lax.bitcast_convert_type(w, jnp.float32)

    out_ref[:, 0:64] = pack(t(ina_ref), t(inc_ref))
    out_ref[:, 64:128] = pack(t(inb_ref), t(ind_ref))


def _repack(tab_t, eye, h, bk):
    n = tab_t.shape[1]
    grid = h // bk
    last_blk = (n - 1) // bk

    def inspec(k):
        return pl.BlockSpec(
            (64, bk),
            lambda ib, g=grid, lb=last_blk, k=k:
            (0, jnp.minimum(ib + k * g, lb)))

    return pl.pallas_call(
        _repack_body,
        grid=(grid,),
        in_specs=[inspec(0), inspec(1), inspec(2), inspec(3),
                  pl.BlockSpec((64, 64), lambda ib: (0, 0))],
        out_specs=pl.BlockSpec((bk, 128), lambda ib: (ib, 0)),
        out_shape=jax.ShapeDtypeStruct((h, 128), jnp.float32),
        compiler_params=pltpu.CompilerParams(
            vmem_limit_bytes=100 * 1024 * 1024),
    )(tab_t, tab_t, tab_t, tab_t, eye)


def _sc_gather_body(n, *args):
    idxs = args[:n]
    pks = args[n:2 * n]
    outs_refs = args[2 * n:3 * n]
    idx_v, q_v, blocks_v, gsem, osem = args[3 * n:]
    wid = lax.axis_index("s") * _NC + lax.axis_index("c")
    row_base = wid * _BPW

    outs = []
    for t, (idx3, pk, out) in enumerate(zip(idxs, pks, outs_refs)):
        h = pk.shape[0]
        copies = []
        for c in range(_NCH):
            pltpu.sync_copy(idx3.at[wid * _NCH + c], idx_v)
            for j in range(_CHUNK // 16):
                v = idx_v[pl.ds(j * 16, 16)]
                q_v[c, pl.ds(j * 16, 16)] = lax.bitwise_and(v, h - 1)
            copies.append(pltpu.async_copy(
                pk.at[q_v.at[c]], blocks_v.at[pl.ds(c * _CHUNK, _CHUNK)],
                gsem))
        for cp in copies:
            cp.wait()
        outs.append(pltpu.async_copy(
            blocks_v, out.at[pl.ds(row_base, _BPW)], osem))
        outs[-1].wait()


def _sc_gather(idx_list, pk_list):
    n = len(idx_list)
    b = idx_list[0].shape[0]
    mesh = plsc.VectorSubcoreMesh(core_axis_name="c", subcore_axis_name="s")
    out_sds = jax.ShapeDtypeStruct((b, 128), jnp.float32)
    fn = pl.kernel(
        functools.partial(_sc_gather_body, n),
        out_type=(out_sds,) * n,
        mesh=mesh,
        scratch_types=[
            pltpu.VMEM((_CHUNK,), jnp.int32),
            pltpu.VMEM((_NCH, _CHUNK), jnp.int32),
            pltpu.VMEM((_BPW, 128), jnp.float32),
            pltpu.SemaphoreType.DMA,
            pltpu.SemaphoreType.DMA,
        ],
    )
    idx2 = [ix.reshape(b // _CHUNK, _CHUNK) for ix in idx_list]
    return fn(*idx2, *pk_list)


def _mlp_body(u_ref, g_ref, i_ref, su_ref, sg_ref, si_ref,
              w1u_ref, w1g_ref, w1i_ref, b1_ref,
              w2_ref, b2_ref, w3_ref, b3_ref, out_ref):
    col = lax.broadcasted_iota(jnp.int32, (1, 128), 1)
    low = col < 64

    def sel(pk_ref, s_ref):
        s = s_ref[...]
        w = lax.bitcast_convert_type(pk_ref[...], jnp.uint32)
        lo = lax.bitcast_convert_type(lax.shift_left(w, jnp.uint32(16)),
                                      jnp.float32)
        hi = lax.bitcast_convert_type(
            lax.bitwise_and(w, jnp.uint32(0xFFFF0000)), jnp.float32)
        v = jnp.where(s >= 1.5, hi, lo)
        odd = jnp.where(s >= 1.5, s - 2.0, s) >= 0.5
        keep = jnp.logical_xor(low, odd)
        return jnp.where(keep, v, 0.0)

    h = (sel(u_ref, su_ref) @ w1u_ref[...]
         + sel(g_ref, sg_ref) @ w1g_ref[...]
         + sel(i_ref, si_ref) @ w1i_ref[...]
         + b1_ref[...])
    h = jnp.maximum(h, 0.0)
    h2 = jnp.maximum(h @ w2_ref[...] + b2_ref[...], 0.0)
    o = h2 @ w3_ref[...] + b3_ref[...]
    out_ref[...] = 1.0 / (1.0 + jnp.exp(-o))


def _mlp(u, g, i, su, sg, si, W1, b1, W2, b2, W3, b3, blk=2048):
    b = u.shape[0]
    grid = b // blk
    w1u = jnp.concatenate([W1[0:64], W1[0:64]], axis=0)
    w1g = jnp.concatenate([W1[64:128], W1[64:128]], axis=0)
    w1i = jnp.concatenate([W1[128:192], W1[128:192]], axis=0)
    row_spec = pl.BlockSpec((blk, 128), lambda ib: (ib, 0))
    s_spec = pl.BlockSpec((blk, 1), lambda ib: (ib, 0))

    def full(a):
        return pl.BlockSpec(a.shape, lambda ib: (0,) * a.ndim)

    return pl.pallas_call(
        _mlp_body,
        grid=(grid,),
        in_specs=[row_spec, row_spec, row_spec,
                  s_spec, s_spec, s_spec,
                  full(w1u), full(w1g), full(w1i), full(b1),
                  full(W2), full(b2), full(W3), full(b3)],
        out_specs=pl.BlockSpec((blk, 1), lambda ib: (ib, 0)),
        out_shape=jax.ShapeDtypeStruct((b, 1), jnp.float32),
    )(u, g, i, su, sg, si, w1u, w1g, w1i, b1, W2, b2, W3, b3)


def kernel(user_input, genre_input, item_input, user_table, genre_table,
           item_table, W1, b1, W2, b2, W3, b3):
    b = user_input.shape[0]
    eye = jnp.eye(64, dtype=jnp.float32)
    upk = _repack(user_table.T, eye, 262144, 8192)
    (u,) = _sc_gather([user_input], [upk])
    ipk = _repack(item_table.T, eye, 262144, 8192)
    gpk = _repack(genre_table.T, eye, 256, 256)
    su = (user_input >> 18).astype(jnp.float32).reshape(b, 1)
    si = (item_input >> 18).astype(jnp.float32).reshape(b, 1)
    sg = (genre_input >> 8).astype(jnp.float32).reshape(b, 1)
    g, i = _sc_gather([genre_input, item_input], [gpk, ipk])
    return _mlp(u, g, i, su, sg, si, W1, b1, W2, b2, W3, b3)
